# Initial kernel scaffold; baseline (speedup 1.0000x reference)
import jax
import jax.numpy as jnp
from jax.experimental import pallas as pl


def kernel(x, table):
    raise NotImplementedError("write your pallas kernel here")


# speedup vs baseline: 4.0854x; 4.0854x over previous
import functools

import jax
import jax.numpy as jnp
from jax import lax
from jax.experimental import pallas as pl
from jax.experimental.pallas import tpu as pltpu
from jax.experimental.pallas import tpu_sc as plsc

D_FEATURE = 64
NUM_WORKERS = 32
CHUNK = 128


@functools.lru_cache(maxsize=None)
def _build(n_total: int, d: int):
    b_per_w = n_total // NUM_WORKERS
    n_chunks = b_per_w // CHUNK
    mesh = plsc.VectorSubcoreMesh(core_axis_name="c", subcore_axis_name="s")

    @functools.partial(
        pl.kernel,
        mesh=mesh,
        out_type=jax.ShapeDtypeStruct((n_total, d), jnp.float32),
        compiler_params=pltpu.CompilerParams(use_tc_tiling_on_sc=False),
        scratch_types=[
            pltpu.VMEM((n_chunks, CHUNK), jnp.int32),
            pltpu.VMEM((CHUNK, d), jnp.float32),
            pltpu.SemaphoreType.DMA,
        ],
    )
    def emb(x_hbm, table_hbm, out_hbm, idx_v, rows_v, gsem):
        wid = lax.axis_index("s") * 2 + lax.axis_index("c")
        base = wid * b_per_w
        pltpu.sync_copy(x_hbm.at[wid], idx_v)

        def body(j, carry):
            pltpu.async_copy(table_hbm.at[idx_v.at[j]], rows_v, gsem).wait()
            pltpu.sync_copy(rows_v, out_hbm.at[pl.ds(base + j * CHUNK, CHUNK)])
            return carry

        lax.fori_loop(0, n_chunks, body, 0)

    return emb


def kernel(x, table):
    b, h = x.shape
    n_total = b * h
    d = table.shape[1]
    x_tiled = x.astype(jnp.int32).reshape(NUM_WORKERS, n_total // (NUM_WORKERS * CHUNK), CHUNK)
    out = _build(n_total, d)(x_tiled, table)
    return out.reshape(b, h, d)


# speedup vs baseline: 4.6168x; 1.1301x over previous
import functools

import jax
import jax.numpy as jnp
from jax import lax
from jax.experimental import pallas as pl
from jax.experimental.pallas import tpu as pltpu
from jax.experimental.pallas import tpu_sc as plsc

D_FEATURE = 64
NUM_WORKERS = 32
CHUNK = 128
NBUF = 4
AHEAD = 2


@functools.lru_cache(maxsize=None)
def _build(n_total: int, d: int):
    b_per_w = n_total // NUM_WORKERS
    n_chunks = b_per_w // CHUNK
    assert (n_chunks - AHEAD) % NBUF == 0 and n_chunks - AHEAD >= NBUF
    mesh = plsc.VectorSubcoreMesh(core_axis_name="c", subcore_axis_name="s")

    @functools.partial(
        pl.kernel,
        mesh=mesh,
        out_type=jax.ShapeDtypeStruct((n_total, d), jnp.float32),
        compiler_params=pltpu.CompilerParams(use_tc_tiling_on_sc=False),
        scratch_types=[
            pltpu.VMEM((n_chunks, CHUNK), jnp.int32),
            pltpu.VMEM((NBUF, CHUNK, d), jnp.float32),
        ]
        + [pltpu.SemaphoreType.DMA] * (2 * NBUF),
    )
    def emb(x_hbm, table_hbm, out_hbm, idx_v, rows_v, *sems):
        gsems, wsems = sems[:NBUF], sems[NBUF:]
        wid = lax.axis_index("s") * 2 + lax.axis_index("c")
        base = wid * b_per_w
        pltpu.sync_copy(x_hbm.at[wid], idx_v)

        def start_gather(j, b):
            pltpu.async_copy(table_hbm.at[idx_v.at[j]], rows_v.at[b], gsems[b])

        def wait_gather(b):
            pltpu.make_async_copy(
                out_hbm.at[pl.ds(base, CHUNK)], rows_v.at[b], gsems[b]
            ).wait()

        def start_writeback(j, b):
            pltpu.async_copy(
                rows_v.at[b], out_hbm.at[pl.ds(base + j * CHUNK, CHUNK)], wsems[b]
            )

        def wait_writeback(b):
            pltpu.make_async_copy(
                rows_v.at[b], out_hbm.at[pl.ds(base, CHUNK)], wsems[b]
            ).wait()

        def chunk_step(j, b, do_wait_wb, do_issue):
            wait_gather(b)
            start_writeback(j, b)
            if do_issue:
                bn = (b + AHEAD) % NBUF
                if do_wait_wb:
                    wait_writeback(bn)
                start_gather(j + AHEAD, bn)

        for j in range(AHEAD):
            start_gather(j, j)
        for j in range(NBUF):
            chunk_step(j, j, do_wait_wb=(j >= NBUF - AHEAD), do_issue=True)

        def outer(o, carry):
            for b in range(NBUF):
                chunk_step(o * NBUF + b, b, do_wait_wb=True, do_issue=True)
            return carry

        lax.fori_loop(1, (n_chunks - AHEAD) // NBUF, outer, 0)

        for j in range(n_chunks - AHEAD, n_chunks):
            chunk_step(j, j % NBUF, do_wait_wb=False, do_issue=False)
        for j in range(n_chunks - NBUF, n_chunks):
            wait_writeback(j % NBUF)

    return emb


def kernel(x, table):
    b, h = x.shape
    n_total = b * h
    d = table.shape[1]
    x_tiled = x.astype(jnp.int32).reshape(
        NUM_WORKERS, n_total // (NUM_WORKERS * CHUNK), CHUNK
    )
    out = _build(n_total, d)(x_tiled, table)
    return out.reshape(b, h, d)


# speedup vs baseline: 4.6664x; 1.0107x over previous
import functools

import jax
import jax.numpy as jnp
from jax import lax
from jax.experimental import pallas as pl
from jax.experimental.pallas import tpu as pltpu
from jax.experimental.pallas import tpu_sc as plsc

D_FEATURE = 64
NUM_WORKERS = 32
CHUNK = 128
NBUF = 8
AHEAD = 4


@functools.lru_cache(maxsize=None)
def _build(n_total: int, d: int):
    b_per_w = n_total // NUM_WORKERS
    n_chunks = b_per_w // CHUNK
    n_steady = (n_chunks - NBUF - AHEAD) // NBUF
    steady_end = NBUF + n_steady * NBUF
    assert n_chunks >= NBUF + AHEAD and AHEAD <= NBUF - AHEAD
    mesh = plsc.VectorSubcoreMesh(core_axis_name="c", subcore_axis_name="s")

    @functools.partial(
        pl.kernel,
        mesh=mesh,
        out_type=jax.ShapeDtypeStruct((n_total, d), jnp.float32),
        compiler_params=pltpu.CompilerParams(use_tc_tiling_on_sc=False),
        scratch_types=[
            pltpu.VMEM((n_chunks, CHUNK), jnp.int32),
            pltpu.VMEM((NBUF, CHUNK, d), jnp.float32),
        ]
        + [pltpu.SemaphoreType.DMA] * (2 * NBUF),
    )
    def emb(x_hbm, table_hbm, out_hbm, idx_v, rows_v, *sems):
        gsems, wsems = sems[:NBUF], sems[NBUF:]
        wid = lax.axis_index("s") * 2 + lax.axis_index("c")
        base = wid * b_per_w
        pltpu.sync_copy(x_hbm.at[wid], idx_v)

        def start_gather(j, b):
            pltpu.async_copy(table_hbm.at[idx_v.at[j]], rows_v.at[b], gsems[b])

        def wait_gather(b):
            pltpu.make_async_copy(
                out_hbm.at[pl.ds(base, CHUNK)], rows_v.at[b], gsems[b]
            ).wait()

        def start_writeback(j, b):
            pltpu.async_copy(
                rows_v.at[b], out_hbm.at[pl.ds(base + j * CHUNK, CHUNK)], wsems[b]
            )

        def wait_writeback(b):
            pltpu.make_async_copy(
                rows_v.at[b], out_hbm.at[pl.ds(base, CHUNK)], wsems[b]
            ).wait()

        def chunk_step(j, b, do_wait_wb, do_issue):
            wait_gather(b)
            start_writeback(j, b)
            if do_issue:
                bn = (b + AHEAD) % NBUF
                if do_wait_wb:
                    wait_writeback(bn)
                start_gather(j + AHEAD, bn)

        for j in range(AHEAD):
            start_gather(j, j)
        for j in range(NBUF):
            chunk_step(j, j, do_wait_wb=(j >= NBUF - AHEAD), do_issue=True)

        def outer(o, carry):
            for b in range(NBUF):
                chunk_step(o * NBUF + b, b, do_wait_wb=True, do_issue=True)
            return carry

        lax.fori_loop(1, 1 + n_steady, outer, 0)

        for j in range(steady_end, n_chunks):
            chunk_step(j, j % NBUF, do_wait_wb=True, do_issue=(j + AHEAD < n_chunks))
        for j in range(n_chunks - NBUF, n_chunks):
            wait_writeback(j % NBUF)

    return emb


def kernel(x, table):
    b, h = x.shape
    n_total = b * h
    d = table.shape[1]
    x_tiled = x.astype(jnp.int32).reshape(
        NUM_WORKERS, n_total // (NUM_WORKERS * CHUNK), CHUNK
    )
    out = _build(n_total, d)(x_tiled, table)
    return out.reshape(b, h, d)
